# baseline (speedup 1.0000x reference)
import functools

import jax
import jax.numpy as jnp
from jax import lax
from jax.experimental import pallas as pl
from jax.experimental.pallas import tpu as pltpu
from jax.experimental.pallas import tpu_sc as plsc


def kernel(x, W, pos_enc):
    B, S = x.shape
    V, D = W.shape
    L = 16
    total = B * S

    info = plsc.get_sparse_core_info()
    NW = info.num_cores * info.num_subcores
    rows_per_w = total // NW
    C = S
    SPLIT = 104
    NB = 4
    n_chunks = rows_per_w // C
    assert total % NW == 0 and rows_per_w % C == 0 and n_chunks % NB == 0
    assert SPLIT % 8 == 0 and SPLIT <= 128 and C - SPLIT <= 128

    x_flat = x.reshape(-1).astype(jnp.int32)
    pos = pos_enc[0, :S, :]

    mesh = plsc.VectorSubcoreMesh(core_axis_name="c", subcore_axis_name="s")

    @functools.partial(
        pl.kernel,
        out_type=jax.ShapeDtypeStruct((total, D), jnp.float32),
        mesh=mesh,
        scratch_types=[pltpu.VMEM((C,), jnp.int32)] * NB
        + [
            pltpu.VMEM((NB, C, D), jnp.float32),
            pltpu.VMEM((C, D), jnp.float32),
        ]
        + [pltpu.SemaphoreType.DMA] * (3 * NB),
    )
    def emb(x_hbm, w_hbm, pos_hbm, out_hbm, *refs):
        idx_v = refs[:NB]
        rows_v, pos_v = refs[NB], refs[NB + 1]
        sems = refs[NB + 2 :]
        isem = sems[:NB]
        gsem = sems[NB : 2 * NB]
        osem = sems[2 * NB :]
        wid = lax.axis_index("s") * info.num_cores + lax.axis_index("c")
        base = wid * rows_per_w
        pltpu.sync_copy(pos_hbm, pos_v)

        def idx_desc(g, b):
            return pltpu.make_async_copy(
                x_hbm.at[pl.ds(base + g * C, C)], idx_v[b], isem[b]
            )

        def gather_descs(b):
            descs = []
            for lo, n in ((0, SPLIT), (SPLIT, C - SPLIT)):
                descs.append(
                    pltpu.make_async_copy(
                        w_hbm.at[idx_v[b].at[pl.ds(lo, n)]],
                        rows_v.at[b, pl.ds(lo, n)],
                        gsem[b],
                    )
                )
            return descs

        def out_desc(g, b):
            return pltpu.make_async_copy(
                rows_v.at[b], out_hbm.at[pl.ds(base + g * C, C)], osem[b]
            )

        IA = 2

        for k in range(IA + 1):
            idx_desc(k, k).start()
        for k in range(IA):
            idx_desc(k, k).wait()
            for d in gather_descs(k):
                d.start()

        def outer(i, carry):
            g0 = i * NB
            for b in range(NB):
                g = g0 + b
                f = g + IA
                bf = (b + IA) % NB

                @pl.when(f < n_chunks)
                def _():
                    @pl.when(g >= NB - IA)
                    def _():
                        out_desc(f - NB, bf).wait()

                    idx_desc(f, bf).wait()
                    for d in gather_descs(bf):
                        d.start()

                @pl.when(f + 1 < n_chunks)
                def _():
                    idx_desc(f + 1, (bf + 1) % NB).start()

                for d, (lo, n) in zip(
                    gather_descs(b), ((0, SPLIT), (SPLIT, C - SPLIT))
                ):
                    d.wait()

                    @plsc.parallel_loop(lo, lo + n, unroll=4)
                    def _(r):
                        for j in range(D // L):
                            sl = pl.ds(j * L, L)
                            rows_v[b, r, sl] = rows_v[b, r, sl] + pos_v[r, sl]

                out_desc(g, b).start()
            return carry

        lax.fori_loop(0, n_chunks // NB, outer, 0)

        for j in range(NB):
            g = n_chunks - NB + j
            out_desc(g, g % NB).wait()

    out = emb(x_flat, W, pos)
    return out.reshape(B, S, D)


# speedup vs baseline: 1.0034x; 1.0034x over previous
import functools

import jax
import jax.numpy as jnp
from jax import lax
from jax.experimental import pallas as pl
from jax.experimental.pallas import tpu as pltpu
from jax.experimental.pallas import tpu_sc as plsc


def kernel(x, W, pos_enc):
    B, S = x.shape
    V, D = W.shape
    L = 16
    total = B * S

    info = plsc.get_sparse_core_info()
    NW = info.num_cores * info.num_subcores
    rows_per_w = total // NW
    C = S
    SPLIT = 104
    NB = 4
    n_chunks = rows_per_w // C
    assert total % NW == 0 and rows_per_w % C == 0 and n_chunks % NB == 0
    assert SPLIT % 8 == 0 and SPLIT <= 128 and C - SPLIT <= 128

    x_flat = x.reshape(-1).astype(jnp.int32)
    pos = pos_enc[0, :S, :]

    mesh = plsc.VectorSubcoreMesh(core_axis_name="c", subcore_axis_name="s")

    @functools.partial(
        pl.kernel,
        out_type=jax.ShapeDtypeStruct((total, D), jnp.float32),
        mesh=mesh,
        scratch_types=[pltpu.VMEM((C,), jnp.int32)] * NB
        + [
            pltpu.VMEM((NB, C, D), jnp.float32),
            pltpu.VMEM((C, D), jnp.float32),
        ]
        + [pltpu.SemaphoreType.DMA] * (3 * NB),
    )
    def emb(x_hbm, w_hbm, pos_hbm, out_hbm, *refs):
        idx_v = refs[:NB]
        rows_v, pos_v = refs[NB], refs[NB + 1]
        sems = refs[NB + 2 :]
        isem = sems[:NB]
        gsem = sems[NB : 2 * NB]
        osem = sems[2 * NB :]
        wid = lax.axis_index("s") * info.num_cores + lax.axis_index("c")
        base = wid * rows_per_w
        pltpu.sync_copy(pos_hbm, pos_v)

        def idx_desc(g, b):
            return pltpu.make_async_copy(
                x_hbm.at[pl.ds(base + g * C, C)], idx_v[b], isem[b]
            )

        def gather_descs(b):
            descs = []
            for lo, n in ((0, SPLIT), (SPLIT, C - SPLIT)):
                descs.append(
                    pltpu.make_async_copy(
                        w_hbm.at[idx_v[b].at[pl.ds(lo, n)]],
                        rows_v.at[b, pl.ds(lo, n)],
                        gsem[b],
                    )
                )
            return descs

        def out_desc(g, b):
            return pltpu.make_async_copy(
                rows_v.at[b], out_hbm.at[pl.ds(base + g * C, C)], osem[b]
            )

        IA = 2

        for k in range(IA + 1):
            idx_desc(k, k).start()
        for k in range(IA):
            idx_desc(k, k).wait()
            for d in gather_descs(k):
                d.start()

        def outer(i, carry):
            g0 = i * NB
            for b in range(NB):
                g = g0 + b
                f = g + IA
                bf = (b + IA) % NB

                @pl.when(f < n_chunks)
                def _():
                    @pl.when(g >= NB - IA)
                    def _():
                        out_desc(f - NB, bf).wait()

                    idx_desc(f, bf).wait()
                    for d in gather_descs(bf):
                        d.start()

                @pl.when(f + 1 < n_chunks)
                def _():
                    idx_desc(f + 1, (bf + 1) % NB).start()

                for d in gather_descs(b):
                    d.wait()

                @plsc.parallel_loop(0, C, unroll=4)
                def _(r):
                    for j in range(D // L):
                        sl = pl.ds(j * L, L)
                        rows_v[b, r, sl] = rows_v[b, r, sl] + pos_v[r, sl]

                out_desc(g, b).start()
            return carry

        lax.fori_loop(0, n_chunks // NB, outer, 0)

        for j in range(NB):
            g = n_chunks - NB + j
            out_desc(g, g % NB).wait()

    out = emb(x_flat, W, pos)
    return out.reshape(B, S, D)
